# baseline (speedup 1.0000x reference)
import jax
import jax.numpy as jnp
from jax.experimental import pallas as pl
from jax.experimental.pallas import tpu as pltpu

E = 8
D = 2048
DFF = 4096
T = 256
NF = 8
BF = DFF // NF
BD = D // NF


def _moe_kernel(xT_ref, wr_ref, br_ref, w1_ref, w2_ref, w3_ref,
                b1_ref, b2_ref, b3_ref, out_ref,
                wrow_ref, h_ref, yacc_ref):
    e = pl.program_id(0)
    f = pl.program_id(1)

    @pl.when((e == 0) & (f == 0))
    def _init():
        logits = jnp.dot(wr_ref[...], xT_ref[...],
                         preferred_element_type=jnp.float32) + br_ref[...]
        idx = jax.lax.broadcasted_iota(jnp.int32, (E, T), 0)
        m1 = jnp.max(logits, axis=0, keepdims=True)
        i1 = jnp.min(jnp.where(logits == m1, idx, E), axis=0, keepdims=True)
        sel1 = idx == i1
        masked = jnp.where(sel1, -jnp.inf, logits)
        m2 = jnp.max(masked, axis=0, keepdims=True)
        i2 = jnp.min(jnp.where(masked == m2, idx, E), axis=0, keepdims=True)
        sel2 = idx == i2
        p1 = 1.0 / (1.0 + jnp.exp(m2 - m1))
        wrow_ref[...] = jnp.where(sel1, p1, 0.0) + jnp.where(sel2, 1.0 - p1, 0.0)

    @pl.when(e < E)
    def _h_phase():
        xv = xT_ref[...]
        b1f = b1_ref[0, pl.ds(f * BF, BF), :]
        b2f = b2_ref[0, pl.ds(f * BF, BF), :]
        h1 = jnp.dot(w1_ref[0], xv, preferred_element_type=jnp.float32) + b1f
        h2 = jnp.dot(w2_ref[0], xv, preferred_element_type=jnp.float32) + b2f
        h = h2 * (h1 * jax.nn.sigmoid(h1))
        h_ref[e % 2, pl.ds(f * BF, BF), :] = h.astype(jnp.bfloat16)

    @pl.when(e > 0)
    def _y_phase():
        ep = e - 1
        hprev = h_ref[(e - 1) % 2]
        yblk = jnp.dot(w3_ref[0], hprev, preferred_element_type=jnp.float32)
        wrow = wrow_ref[pl.ds(ep, 1), :]
        b3f = b3_ref[0, pl.ds(f * BD, BD), :]
        contrib = (yblk + b3f) * wrow

        @pl.when(ep == 0)
        def _first():
            yacc_ref[pl.ds(f * BD, BD), :] = contrib

        @pl.when(ep > 0)
        def _rest():
            yacc_ref[pl.ds(f * BD, BD), :] += contrib

        @pl.when((e == E) & (f == NF - 1))
        def _emit():
            out_ref[...] = yacc_ref[...]


def kernel(x, Wr, br, W1, b1, W2, b2, W3, b3):
    b, s, d = x.shape
    xT = x.reshape(b * s, d).T
    last = E - 1

    def w12_map(e, f):
        ec = jnp.minimum(e, last)
        fc = jnp.where(e == E, NF - 1, f)
        return (ec, fc, 0)

    def w3_map(e, f):
        ep = jnp.maximum(e - 1, 0)
        fc = jnp.where(e == 0, 0, f)
        return (ep, fc, 0)

    outT = pl.pallas_call(
        _moe_kernel,
        grid=(E + 1, NF),
        in_specs=[
            pl.BlockSpec((D, T), lambda e, f: (0, 0)),
            pl.BlockSpec((E, D), lambda e, f: (0, 0)),
            pl.BlockSpec((E, 1), lambda e, f: (0, 0)),
            pl.BlockSpec((1, BF, D), w12_map),
            pl.BlockSpec((1, BF, D), w12_map),
            pl.BlockSpec((1, BD, DFF), w3_map),
            pl.BlockSpec((1, DFF, 1), lambda e, f: (jnp.minimum(e, last), 0, 0)),
            pl.BlockSpec((1, DFF, 1), lambda e, f: (jnp.minimum(e, last), 0, 0)),
            pl.BlockSpec((1, D, 1), lambda e, f: (jnp.maximum(e - 1, 0), 0, 0)),
        ],
        out_specs=pl.BlockSpec((D, T), lambda e, f: (0, 0)),
        out_shape=jax.ShapeDtypeStruct((D, T), jnp.float32),
        scratch_shapes=[
            pltpu.VMEM((E, T), jnp.float32),
            pltpu.VMEM((2, DFF, T), jnp.bfloat16),
            pltpu.VMEM((D, T), jnp.float32),
        ],
    )(xT, Wr, br.reshape(E, 1), W1, W2, W3,
      b1.reshape(E, DFF, 1), b2.reshape(E, DFF, 1), b3.reshape(E, D, 1))
    return outT.T.reshape(b, s, d)
